# baseline (speedup 1.0000x reference)
import jax
import jax.numpy as jnp
from jax import lax
from jax.experimental import pallas as pl
from jax.experimental.pallas import tpu as pltpu
from jax.experimental.pallas import tpu_sc as plsc

D_MODEL = 64
N_ROWS = 4096
SEQ = 200
B = N_ROWS * SEQ
NC, NS = 2, 16
NW = NC * NS
B_PER_W = B // NW
CH = 64
NCHUNK = B_PER_W // CH
TB = SEQ // 8
NBUF = 4
LAG = 2
SCALE = 8.0

_mesh = plsc.VectorSubcoreMesh(core_axis_name="c", subcore_axis_name="s")


def _sc_body(idx_hbm, table_hbm, out_hbm, idx_v, r0, r1, r2, r3,
             b0, b1, b2, b3, g0, g1, g2, g3, o0, o1, o2, o3):
    rows = (r0, r1, r2, r3)
    obuf = (b0, b1, b2, b3)
    gsem = (g0, g1, g2, g3)
    osem = (o0, o1, o2, o3)
    wid = lax.axis_index("s") * NC + lax.axis_index("c")

    pltpu.sync_copy(idx_hbm.at[wid], idx_v)

    def out_ref(j):
        return out_hbm.at[wid * 16 + j // TB, pl.ds((j % TB) * 4, 4)]

    def gather(j, b):
        return pltpu.make_async_copy(table_hbm.at[idx_v.at[j]], rows[b],
                                     gsem[b])

    def out_copy(j, b):
        return pltpu.make_async_copy(obuf[b], out_ref(j), osem[b])

    def scale_repack(b):
        src, dst = rows[b], obuf[b]

        @pl.loop(0, CH)
        def _(k):
            a8 = k >> 3
            t8 = k & 7
            cl = t8 >> 1
            l0 = (t8 & 1) * D_MODEL
            for c in range(D_MODEL // 16):
                dst[cl, a8, pl.ds(l0 + c * 16, 16)] = (
                    src[k, pl.ds(c * 16, 16)] * SCALE)

    for j in range(LAG):
        gather(j, j % NBUF).start()

    @pl.loop(0, NCHUNK // NBUF)
    def _(g):
        j0 = g * NBUF
        for b in range(NBUF):
            j = j0 + b
            bp = (b + LAG) % NBUF

            @pl.when(j >= LAG)
            def _():
                out_copy(j - LAG, bp).wait()

            @pl.when(j + LAG < NCHUNK)
            def _():
                gather(j + LAG, bp).start()

            gather(j, b).wait()
            scale_repack(b)
            out_copy(j, b).start()

    for j in range(NCHUNK - LAG, NCHUNK):
        out_copy(j, j % NBUF).wait()


_sc_call = pl.kernel(
    _sc_body,
    out_type=jax.ShapeDtypeStruct((N_ROWS // 8, SEQ * D_MODEL // 128, 8, 128),
                                  jnp.float32),
    mesh=_mesh,
    compiler_params=pltpu.CompilerParams(use_tc_tiling_on_sc=False),
    scratch_types=[
        pltpu.VMEM((NCHUNK, CH), jnp.int32),
        pltpu.VMEM((CH, D_MODEL), jnp.float32),
        pltpu.VMEM((CH, D_MODEL), jnp.float32),
        pltpu.VMEM((CH, D_MODEL), jnp.float32),
        pltpu.VMEM((CH, D_MODEL), jnp.float32),
        pltpu.VMEM((4, 8, 128), jnp.float32),
        pltpu.VMEM((4, 8, 128), jnp.float32),
        pltpu.VMEM((4, 8, 128), jnp.float32),
        pltpu.VMEM((4, 8, 128), jnp.float32),
        pltpu.SemaphoreType.DMA,
        pltpu.SemaphoreType.DMA,
        pltpu.SemaphoreType.DMA,
        pltpu.SemaphoreType.DMA,
        pltpu.SemaphoreType.DMA,
        pltpu.SemaphoreType.DMA,
        pltpu.SemaphoreType.DMA,
        pltpu.SemaphoreType.DMA,
    ],
)


def kernel(x, table):
    xw = (x.astype(jnp.int32)
          .reshape(NW, 16, 8, TB, 8)
          .transpose(0, 1, 3, 2, 4)
          .reshape(NW, NCHUNK, CH))
    tbl_pinned = lax.optimization_barrier(table)
    tbl2 = lax.optimization_barrier(tbl_pinned.reshape(500000, 2 * D_MODEL))
    tbl = tbl2.reshape(1000000, D_MODEL)
    out4 = lax.optimization_barrier(_sc_call(xw, tbl))
    return out4.transpose(0, 2, 1, 3).reshape(N_ROWS, SEQ, D_MODEL)
